# baseline (speedup 1.0000x reference)
import functools

import numpy as np
import jax
import jax.numpy as jnp
from jax import lax
from jax.experimental import pallas as pl
from jax.experimental.pallas import tpu as pltpu
from jax.experimental.pallas import tpu_sc as plsc

N = 10000
E = 160000
B = 64
FEAT = 128
NPTS = 128
LOG_SIGMA = float(np.log(25.0))


PN_NODES = 80


def _pointnet_body(pcs_ref, p1_ref, pb1_ref, p2_ref, pb2_ref, p3_ref, pb3_ref,
                   out_ref):
    p = pcs_ref[...]
    h = jnp.maximum(
        jnp.dot(p, p1_ref[...], preferred_element_type=jnp.float32)
        + pb1_ref[...], 0.0)
    h = jnp.maximum(
        jnp.dot(h, p2_ref[...], preferred_element_type=jnp.float32)
        + pb2_ref[...], 0.0)
    h = jnp.dot(h, p3_ref[...], preferred_element_type=jnp.float32) + pb3_ref[...]
    h = h.reshape(PN_NODES, NPTS, FEAT)
    out_ref[...] = jnp.max(h, axis=1)


def _pointnet(pcs_flat, P1, pb1, P2, pb2, P3, pb3):
    grid = N // PN_NODES
    return pl.pallas_call(
        _pointnet_body,
        grid=(grid,),
        in_specs=[
            pl.BlockSpec((PN_NODES * NPTS, 3), lambda i: (i, 0)),
            pl.BlockSpec((3, 64), lambda i: (0, 0)),
            pl.BlockSpec((1, 64), lambda i: (0, 0)),
            pl.BlockSpec((64, 128), lambda i: (0, 0)),
            pl.BlockSpec((1, 128), lambda i: (0, 0)),
            pl.BlockSpec((128, FEAT), lambda i: (0, 0)),
            pl.BlockSpec((1, FEAT), lambda i: (0, 0)),
        ],
        out_specs=pl.BlockSpec((PN_NODES, FEAT), lambda i: (i, 0)),
        out_shape=jax.ShapeDtypeStruct((N, FEAT), jnp.float32),
    )(pcs_flat, P1, pb1, P2, pb2, P3, pb3)


NB = 1000


def _swish(t):
    return t / (1.0 + jnp.exp(-t))


def _node0_body(x_ref, z_ref, batch_ref, rt_ref, we_ref, be_ref, fw_ref,
                tw_ref, tb_ref, w1_ref, b1_ref, emb_ref,
                a_ref, c_ref, te_ref, std_ref):
    bvals = batch_ref[...]
    iota = lax.broadcasted_iota(jnp.int32, (NB, B), 1)
    onehot = (bvals == iota).astype(jnp.float32)
    t = jnp.dot(onehot, rt_ref[...], preferred_element_type=jnp.float32)
    std = jnp.sqrt((jnp.exp(t * (2.0 * LOG_SIGMA)) - 1.0) / (2.0 * LOG_SIGMA))
    xn = x_ref[...] + z_ref[...] * std
    h0 = xn * we_ref[...] + be_ref[...]
    proj = t * fw_ref[...] * (2.0 * np.pi)
    tw = tw_ref[...]
    te = (jnp.dot(jnp.sin(proj), tw[:64, :], preferred_element_type=jnp.float32)
          + jnp.dot(jnp.cos(proj), tw[64:, :], preferred_element_type=jnp.float32)
          + tb_ref[...])
    te = _swish(te)
    w1 = w1_ref[...]
    wd = w1[:384, :] - w1[384:, :]
    wl = w1[384:, :]
    emb = emb_ref[...]
    a = (jnp.dot(h0, wd[:128, :], preferred_element_type=jnp.float32)
         + jnp.dot(te, wd[128:256, :], preferred_element_type=jnp.float32)
         + jnp.dot(emb, wd[256:, :], preferred_element_type=jnp.float32)
         + b1_ref[...])
    c = (jnp.dot(h0, wl[:128, :], preferred_element_type=jnp.float32)
         + jnp.dot(te, wl[128:256, :], preferred_element_type=jnp.float32)
         + jnp.dot(emb, wl[256:, :], preferred_element_type=jnp.float32))
    a_ref[...] = a
    c_ref[...] = c
    te_ref[...] = te
    std_ref[...] = std


def _node0(x, z, batch2, rt, We, be2, fW2, tW, tb2, W1, b12, emb):
    grid = N // NB
    nspec = lambda w: pl.BlockSpec((NB, w), lambda i: (i, 0))
    full = lambda r, c: pl.BlockSpec((r, c), lambda i: (0, 0))
    return pl.pallas_call(
        _node0_body,
        grid=(grid,),
        in_specs=[
            nspec(1), nspec(1), nspec(1),
            full(B, 1), full(1, 128), full(1, 128), full(1, 64),
            full(128, 128), full(1, 128), full(768, 128), full(1, 128),
            nspec(128),
        ],
        out_specs=[nspec(128), nspec(128), nspec(128), nspec(1)],
        out_shape=[
            jax.ShapeDtypeStruct((N, FEAT), jnp.float32),
            jax.ShapeDtypeStruct((N, FEAT), jnp.float32),
            jax.ShapeDtypeStruct((N, FEAT), jnp.float32),
            jax.ShapeDtypeStruct((N, 1), jnp.float32),
        ],
    )(x, z, batch2, rt, We, be2, fW2, tW, tb2, W1, b12, emb)


def _nodel_body(m_ref, b2p_ref, te_ref, emb_ref, w1_ref, b1_ref, a_ref, c_ref):
    m = m_ref[...]
    h = jnp.maximum(jnp.where(jnp.isfinite(m), m + b2p_ref[...], 0.0), 0.0)
    w1 = w1_ref[...]
    wd = w1[:384, :] - w1[384:, :]
    wl = w1[384:, :]
    te = te_ref[...]
    emb = emb_ref[...]
    a = (jnp.dot(h, wd[:128, :], preferred_element_type=jnp.float32)
         + jnp.dot(te, wd[128:256, :], preferred_element_type=jnp.float32)
         + jnp.dot(emb, wd[256:, :], preferred_element_type=jnp.float32)
         + b1_ref[...])
    c = (jnp.dot(h, wl[:128, :], preferred_element_type=jnp.float32)
         + jnp.dot(te, wl[128:256, :], preferred_element_type=jnp.float32)
         + jnp.dot(emb, wl[256:, :], preferred_element_type=jnp.float32))
    a_ref[...] = a
    c_ref[...] = c


def _nodel(M, b2p2, te, emb, W1, b12):
    grid = N // NB
    nspec = lambda w: pl.BlockSpec((NB, w), lambda i: (i, 0))
    full = lambda r, c: pl.BlockSpec((r, c), lambda i: (0, 0))
    return pl.pallas_call(
        _nodel_body,
        grid=(grid,),
        in_specs=[nspec(128), full(1, 128), nspec(128), nspec(128),
                  full(768, 128), full(1, 128)],
        out_specs=[nspec(128), nspec(128)],
        out_shape=[jax.ShapeDtypeStruct((N, FEAT), jnp.float32),
                   jax.ShapeDtypeStruct((N, FEAT), jnp.float32)],
    )(M, b2p2, te, emb, W1, b12)


NW = 32
EW = E // NW
CH = 200


def _gather_rows(table, idx):
    mesh = plsc.VectorSubcoreMesh(core_axis_name="c", subcore_axis_name="s")

    @functools.partial(
        pl.kernel,
        mesh=mesh,
        out_type=jax.ShapeDtypeStruct((E, FEAT), jnp.float32),
        scratch_types=[
            pltpu.VMEM((CH,), jnp.int32),
            pltpu.VMEM((CH, FEAT), jnp.float32),
            pltpu.SemaphoreType.DMA,
        ],
    )
    def k(table_hbm, idx_hbm, out_hbm, idxv, rowsv, sem):
        wid = lax.axis_index("s") * 2 + lax.axis_index("c")

        def chunk(c, carry):
            base = wid * EW + c * CH
            pltpu.sync_copy(idx_hbm.at[pl.ds(base, CH)], idxv)
            pltpu.async_copy(table_hbm.at[idxv], rowsv, sem).wait()
            pltpu.sync_copy(rowsv, out_hbm.at[pl.ds(base, CH)])
            return carry

        lax.fori_loop(0, EW // CH, chunk, 0)

    return k(table, idx)


BLKE = 2000


def _edge_body(ga_ref, gc_ref, w2_ref, out_ref):
    pre = jnp.maximum(ga_ref[...] + gc_ref[...], 0.0)
    out_ref[...] = jnp.dot(pre, w2_ref[...], preferred_element_type=jnp.float32)


def _edgemm(GA, GC, W2):
    grid = E // BLKE
    espec = pl.BlockSpec((BLKE, FEAT), lambda i: (i, 0))
    return pl.pallas_call(
        _edge_body,
        grid=(grid,),
        in_specs=[espec, espec, pl.BlockSpec((FEAT, FEAT), lambda i: (0, 0))],
        out_specs=espec,
        out_shape=jax.ShapeDtypeStruct((E, FEAT), jnp.float32),
    )(GA, GC, W2)


def _segmax_body(dst_ref, m_ref, out_ref):
    @pl.when(pl.program_id(0) == 0)
    def _():
        out_ref[...] = jnp.full((N, FEAT), -jnp.inf, jnp.float32)

    def step(i, carry):
        d = dst_ref[0, 0, i]
        out_ref[pl.ds(d, 1), :] = jnp.maximum(out_ref[pl.ds(d, 1), :],
                                              m_ref[pl.ds(i, 1), :])
        return carry

    lax.fori_loop(0, BLKE, step, 0)


def _segmax(dst3, m):
    grid = E // BLKE
    return pl.pallas_call(
        _segmax_body,
        grid=(grid,),
        in_specs=[
            pl.BlockSpec((1, 1, BLKE), lambda i: (i, 0, 0),
                         memory_space=pltpu.SMEM),
            pl.BlockSpec((BLKE, FEAT), lambda i: (i, 0)),
        ],
        out_specs=pl.BlockSpec((N, FEAT), lambda i: (0, 0)),
        out_shape=jax.ShapeDtypeStruct((N, FEAT), jnp.float32),
    )(dst3, m)


def _final_body(mc_ref, std_ref, z_ref, b2c_ref, out_ref):
    @pl.when(pl.program_id(0) == 0)
    def _():
        out_ref[0, 0] = 0.0

    m0 = mc_ref[:, 0:1]
    s = jnp.where(jnp.isfinite(m0), m0 + b2c_ref[0, 0], 0.0)
    score = s / (std_ref[...] + 1e-7)
    v = (score * std_ref[...] + z_ref[...]) ** 2
    out_ref[0, 0] += jnp.sum(v)

    @pl.when(pl.program_id(0) == pl.num_programs(0) - 1)
    def _():
        out_ref[0, 0] = out_ref[0, 0] * (1.0 / B)


def _final(Mc, std, z, b2c2):
    grid = N // NB
    return pl.pallas_call(
        _final_body,
        grid=(grid,),
        in_specs=[
            pl.BlockSpec((NB, FEAT), lambda i: (i, 0)),
            pl.BlockSpec((NB, 1), lambda i: (i, 0)),
            pl.BlockSpec((NB, 1), lambda i: (i, 0)),
            pl.BlockSpec((1, 1), lambda i: (0, 0), memory_space=pltpu.SMEM),
        ],
        out_specs=pl.BlockSpec((1, 1), lambda i: (0, 0),
                               memory_space=pltpu.SMEM),
        out_shape=jax.ShapeDtypeStruct((1, 1), jnp.float32),
    )(Mc, std, z, b2c2)


def kernel(x, edge_index, batch, pcs, random_t, z, We, be, W1a, b1a, W2a, b2a,
           W1b, b1b, W2b, b2b, W1c, b1c, W2c, b2c, fW, tW, tb, P1, pb1, P2,
           pb2, P3, pb3):
    src = edge_index[0]
    dst = edge_index[1]
    dst3 = dst.reshape(E // BLKE, 1, BLKE)
    batch2 = batch.reshape(N, 1)
    pcs_flat = pcs.reshape(N * NPTS, 3)
    r = lambda v: v.reshape(1, -1)

    emb = _pointnet(pcs_flat, P1, r(pb1), P2, r(pb2), P3, r(pb3))
    A, C, te, std = _node0(x, z, batch2, random_t, We, r(be), r(fW), tW,
                           r(tb), W1a, r(b1a), emb)

    for W1n, b1n, W2, b2 in ((W1b, b1b, W2a, b2a), (W1c, b1c, W2b, b2b)):
        GA = _gather_rows(A, dst)
        GC = _gather_rows(C, src)
        m = _edgemm(GA, GC, W2)
        M = _segmax(dst3, m)
        A, C = _nodel(M, r(b2), te, emb, W1n, r(b1n))

    GA = _gather_rows(A, dst)
    GC = _gather_rows(C, src)
    W2c_pad = jnp.pad(W2c, ((0, 0), (0, FEAT - W2c.shape[1])))
    mc = _edgemm(GA, GC, W2c_pad)
    Mc = _segmax(dst3, mc)
    loss = _final(Mc, std, z, b2c.reshape(1, 1))
    return loss.reshape(())


# speedup vs baseline: 1.3134x; 1.3134x over previous
import functools

import numpy as np
import jax
import jax.numpy as jnp
from jax import lax
from jax.experimental import pallas as pl
from jax.experimental.pallas import tpu as pltpu
from jax.experimental.pallas import tpu_sc as plsc

N = 10000
E = 160000
B = 64
FEAT = 128
NPTS = 128
LOG_SIGMA = float(np.log(25.0))


PN_NODES = 80


def _pointnet_body(pcs_ref, p1_ref, pb1_ref, p2_ref, pb2_ref, p3_ref, pb3_ref,
                   out_ref):
    p = pcs_ref[...]
    h = jnp.maximum(
        jnp.dot(p, p1_ref[...], preferred_element_type=jnp.float32)
        + pb1_ref[...], 0.0)
    h = jnp.maximum(
        jnp.dot(h, p2_ref[...], preferred_element_type=jnp.float32)
        + pb2_ref[...], 0.0)
    h = jnp.dot(h, p3_ref[...], preferred_element_type=jnp.float32) + pb3_ref[...]
    h = h.reshape(PN_NODES, NPTS, FEAT)
    out_ref[...] = jnp.max(h, axis=1)


def _pointnet(pcs_flat, P1, pb1, P2, pb2, P3, pb3):
    grid = N // PN_NODES
    return pl.pallas_call(
        _pointnet_body,
        grid=(grid,),
        in_specs=[
            pl.BlockSpec((PN_NODES * NPTS, 3), lambda i: (i, 0)),
            pl.BlockSpec((3, 64), lambda i: (0, 0)),
            pl.BlockSpec((1, 64), lambda i: (0, 0)),
            pl.BlockSpec((64, 128), lambda i: (0, 0)),
            pl.BlockSpec((1, 128), lambda i: (0, 0)),
            pl.BlockSpec((128, FEAT), lambda i: (0, 0)),
            pl.BlockSpec((1, FEAT), lambda i: (0, 0)),
        ],
        out_specs=pl.BlockSpec((PN_NODES, FEAT), lambda i: (i, 0)),
        out_shape=jax.ShapeDtypeStruct((N, FEAT), jnp.float32),
    )(pcs_flat, P1, pb1, P2, pb2, P3, pb3)


NB = 1000


def _swish(t):
    return t / (1.0 + jnp.exp(-t))


def _node0_body(x_ref, z_ref, batch_ref, rt_ref, we_ref, be_ref, fw_ref,
                tw_ref, tb_ref, w1_ref, b1_ref, emb_ref,
                a_ref, c_ref, te_ref, std_ref):
    bvals = batch_ref[...]
    iota = lax.broadcasted_iota(jnp.int32, (NB, B), 1)
    onehot = (bvals == iota).astype(jnp.float32)
    t = jnp.dot(onehot, rt_ref[...], preferred_element_type=jnp.float32)
    std = jnp.sqrt((jnp.exp(t * (2.0 * LOG_SIGMA)) - 1.0) / (2.0 * LOG_SIGMA))
    xn = x_ref[...] + z_ref[...] * std
    h0 = xn * we_ref[...] + be_ref[...]
    proj = t * fw_ref[...] * (2.0 * np.pi)
    tw = tw_ref[...]
    te = (jnp.dot(jnp.sin(proj), tw[:64, :], preferred_element_type=jnp.float32)
          + jnp.dot(jnp.cos(proj), tw[64:, :], preferred_element_type=jnp.float32)
          + tb_ref[...])
    te = _swish(te)
    w1 = w1_ref[...]
    wd = w1[:384, :] - w1[384:, :]
    wl = w1[384:, :]
    emb = emb_ref[...]
    a = (jnp.dot(h0, wd[:128, :], preferred_element_type=jnp.float32)
         + jnp.dot(te, wd[128:256, :], preferred_element_type=jnp.float32)
         + jnp.dot(emb, wd[256:, :], preferred_element_type=jnp.float32)
         + b1_ref[...])
    c = (jnp.dot(h0, wl[:128, :], preferred_element_type=jnp.float32)
         + jnp.dot(te, wl[128:256, :], preferred_element_type=jnp.float32)
         + jnp.dot(emb, wl[256:, :], preferred_element_type=jnp.float32))
    a_ref[...] = a
    c_ref[...] = c
    te_ref[...] = te
    std_ref[...] = std


def _node0(x, z, batch2, rt, We, be2, fW2, tW, tb2, W1, b12, emb):
    grid = N // NB
    nspec = lambda w: pl.BlockSpec((NB, w), lambda i: (i, 0))
    full = lambda r, c: pl.BlockSpec((r, c), lambda i: (0, 0))
    return pl.pallas_call(
        _node0_body,
        grid=(grid,),
        in_specs=[
            nspec(1), nspec(1), nspec(1),
            full(B, 1), full(1, 128), full(1, 128), full(1, 64),
            full(128, 128), full(1, 128), full(768, 128), full(1, 128),
            nspec(128),
        ],
        out_specs=[nspec(128), nspec(128), nspec(128), nspec(1)],
        out_shape=[
            jax.ShapeDtypeStruct((N, FEAT), jnp.float32),
            jax.ShapeDtypeStruct((N, FEAT), jnp.float32),
            jax.ShapeDtypeStruct((N, FEAT), jnp.float32),
            jax.ShapeDtypeStruct((N, 1), jnp.float32),
        ],
    )(x, z, batch2, rt, We, be2, fW2, tW, tb2, W1, b12, emb)


def _nodel_body(m_ref, b2p_ref, te_ref, emb_ref, w1_ref, b1_ref, a_ref, c_ref):
    m = m_ref[...]
    h = jnp.maximum(jnp.where(jnp.isfinite(m), m + b2p_ref[...], 0.0), 0.0)
    w1 = w1_ref[...]
    wd = w1[:384, :] - w1[384:, :]
    wl = w1[384:, :]
    te = te_ref[...]
    emb = emb_ref[...]
    a = (jnp.dot(h, wd[:128, :], preferred_element_type=jnp.float32)
         + jnp.dot(te, wd[128:256, :], preferred_element_type=jnp.float32)
         + jnp.dot(emb, wd[256:, :], preferred_element_type=jnp.float32)
         + b1_ref[...])
    c = (jnp.dot(h, wl[:128, :], preferred_element_type=jnp.float32)
         + jnp.dot(te, wl[128:256, :], preferred_element_type=jnp.float32)
         + jnp.dot(emb, wl[256:, :], preferred_element_type=jnp.float32))
    a_ref[...] = a
    c_ref[...] = c


def _nodel(M, b2p2, te, emb, W1, b12):
    grid = N // NB
    nspec = lambda w: pl.BlockSpec((NB, w), lambda i: (i, 0))
    full = lambda r, c: pl.BlockSpec((r, c), lambda i: (0, 0))
    return pl.pallas_call(
        _nodel_body,
        grid=(grid,),
        in_specs=[nspec(128), full(1, 128), nspec(128), nspec(128),
                  full(768, 128), full(1, 128)],
        out_specs=[nspec(128), nspec(128)],
        out_shape=[jax.ShapeDtypeStruct((N, FEAT), jnp.float32),
                   jax.ShapeDtypeStruct((N, FEAT), jnp.float32)],
    )(M, b2p2, te, emb, W1, b12)


NW = 32
EW = E // NW
CH = 200


def _gather_rows(table, idx):
    mesh = plsc.VectorSubcoreMesh(core_axis_name="c", subcore_axis_name="s")

    @functools.partial(
        pl.kernel,
        mesh=mesh,
        out_type=jax.ShapeDtypeStruct((E, FEAT), jnp.float32),
        scratch_types=[
            pltpu.VMEM((CH,), jnp.int32),
            pltpu.VMEM((CH, FEAT), jnp.float32),
            pltpu.SemaphoreType.DMA,
        ],
    )
    def k(table_hbm, idx_hbm, out_hbm, idxv, rowsv, sem):
        wid = lax.axis_index("s") * 2 + lax.axis_index("c")

        def chunk(c, carry):
            base = wid * EW + c * CH
            pltpu.sync_copy(idx_hbm.at[pl.ds(base, CH)], idxv)
            pltpu.async_copy(table_hbm.at[idxv], rowsv, sem).wait()
            pltpu.sync_copy(rowsv, out_hbm.at[pl.ds(base, CH)])
            return carry

        lax.fori_loop(0, EW // CH, chunk, 0)

    return k(table, idx)


NPB = 313
HALF = E // 2
EPAD = E + 512


def _vgather(v, idx):
    return lax.gather(
        v, idx[:, None],
        lax.GatherDimensionNumbers(offset_dims=(), collapsed_slice_dims=(0,),
                                   start_index_map=(0,)),
        (1,), mode=lax.GatherScatterMode.PROMISE_IN_BOUNDS)


def _divnpb(d):
    return lax.shift_right_logical(d * 26802, 23)


def _prefix16(v):
    iota = lax.iota(jnp.int32, 16)
    for s in (1, 2, 4, 8):
        sh = _vgather(v, jnp.maximum(iota - s, 0))
        v = v + jnp.where(iota >= s, sh, 0)
    return v


def _bucket_partition(dst, src):
    mesh = plsc.VectorSubcoreMesh(core_axis_name="c", subcore_axis_name="s")

    @functools.partial(
        pl.kernel,
        mesh=mesh,
        out_type=(
            jax.ShapeDtypeStruct((EPAD,), jnp.int32),
            jax.ShapeDtypeStruct((EPAD,), jnp.int32),
            jax.ShapeDtypeStruct((1056,), jnp.int32),
        ),
        scratch_types=[
            pltpu.VMEM((5120,), jnp.int32),
            pltpu.VMEM((5120,), jnp.int32),
            pltpu.VMEM((5120,), jnp.int32),
            pltpu.VMEM((32,), jnp.int32),
            pltpu.VMEM((512,), jnp.int32),
            pltpu.VMEM((528,), jnp.int32),
            pltpu.VMEM_SHARED((512,), jnp.int32),
            pltpu.SemaphoreType.DMA,
        ],
    )
    def k(dst_hbm, src_hbm, srcs_hbm, dsts_hbm, bb_hbm,
          dbuf, sbuf, lrank, hist, gridflat, bbv, shared, sem):
        cid = lax.axis_index("c")
        sid = lax.axis_index("s")
        wid = cid * 16 + sid
        iota = lax.iota(jnp.int32, 16)
        ew = E // NW
        base0 = wid * ew

        pltpu.sync_copy(dst_hbm.at[pl.ds(base0, ew)], dbuf.at[pl.ds(0, ew)])
        pltpu.sync_copy(src_hbm.at[pl.ds(base0, ew)], sbuf.at[pl.ds(0, ew)])

        z16 = jnp.zeros((16,), jnp.int32)

        def avloop(v, carry):
            c0, c1 = carry
            vb = v * 16
            dv = dbuf[pl.ds(vb, 16)]
            validv = (vb + iota) < ew
            bv = jnp.where(validv, _divnpb(dv), 32)
            r = z16
            for s in range(1, 16):
                sh = _vgather(bv, jnp.maximum(iota - s, 0))
                r = r + jnp.where((sh == bv) & (iota >= s), 1, 0)
            hb = jnp.where(bv < 16,
                           _vgather(c0, jnp.minimum(bv, 15)),
                           _vgather(c1, jnp.clip(bv - 16, 0, 15)))
            lrank[pl.ds(vb, 16)] = hb + r
            add0 = z16
            add1 = z16
            for s in range(16):
                rot = _vgather(bv, (iota + s) & 15)
                add0 = add0 + jnp.where(rot == iota, 1, 0)
                add1 = add1 + jnp.where(rot == iota + 16, 1, 0)
            return (c0 + add0, c1 + add1)

        c0, c1 = lax.fori_loop(0, 320, avloop, (z16, z16))
        hist[pl.ds(0, 16)] = c0
        hist[pl.ds(16, 16)] = c1

        pltpu.sync_copy(hist, shared.at[pl.ds(sid * 32, 32)])
        plsc.subcore_barrier()
        pltpu.sync_copy(shared, gridflat)

        def rowloop(t, carry):
            t0, t1, p0, p1 = carry
            row0 = gridflat[pl.ds(t * 32, 16)]
            row1 = gridflat[pl.ds(t * 32 + 16, 16)]
            before = t < sid
            return (t0 + row0, t1 + row1,
                    p0 + jnp.where(before, row0, 0),
                    p1 + jnp.where(before, row1, 0))

        t0, t1, p0, p1 = lax.fori_loop(0, 16, rowloop, (z16, z16, z16, z16))
        i0 = _prefix16(t0)
        i1 = _prefix16(t1)
        tot0 = _vgather(i0, iota * 0 + 15)
        ex0 = i0 - t0
        ex1 = i1 - t1 + tot0
        tb0 = ex0 + p0
        tb1 = ex1 + p1

        @pl.when(sid == 0)
        def _():
            for q in range(16):
                cq = iota * 0 + q
                bbv[pl.ds(q * 16, 16)] = _vgather(ex0, cq)
                bbv[pl.ds((q + 16) * 16, 16)] = _vgather(ex1, cq)
            bbv[pl.ds(32 * 16, 16)] = iota * 0 + HALF
            pltpu.sync_copy(bbv.at[pl.ds(0, 528)],
                            bb_hbm.at[pl.ds(cid * 528, 528)])

        hbase = cid * HALF
        sink = E + wid * 16 + iota

        def cgroup(g, carry):
            copies = []
            for u in range(16):
                vb = (g * 16 + u) * 16
                dv = dbuf[pl.ds(vb, 16)]
                lr = lrank[pl.ds(vb, 16)]
                validv = (vb + iota) < ew
                bv = jnp.where(validv, _divnpb(dv), 32)
                tb = jnp.where(bv < 16,
                               _vgather(tb0, jnp.minimum(bv, 15)),
                               _vgather(tb1, jnp.clip(bv - 16, 0, 15)))
                pos = jnp.where(validv, hbase + tb + lr, sink)
                copies.append(pltpu.async_copy(
                    sbuf.at[pl.ds(vb, 16)], srcs_hbm.at[pos], sem))
                copies.append(pltpu.async_copy(
                    dbuf.at[pl.ds(vb, 16)], dsts_hbm.at[pos], sem))
            for cp in copies:
                cp.wait()
            return carry

        lax.fori_loop(0, 20, cgroup, 0)

    return k(dst, src)


SCH = 272
SSTEP = 256


def _sc_segmax(m_flat, dsts_perm, bb):
    mesh = plsc.VectorSubcoreMesh(core_axis_name="c", subcore_axis_name="s")

    @functools.partial(
        pl.kernel,
        mesh=mesh,
        out_type=jax.ShapeDtypeStruct((32 * NPB * FEAT,), jnp.float32),
        scratch_types=[
            pltpu.VMEM(((NPB + 1) * FEAT,), jnp.float32),
            pltpu.VMEM((SCH * FEAT,), jnp.float32),
            pltpu.VMEM((SCH,), jnp.int32),
            pltpu.VMEM((32,), jnp.int32),
        ],
    )
    def k(m_hbm, dst_hbm, bb_hbm, out_hbm, acc, mbuf, dbuf, bbv):
        cid = lax.axis_index("c")
        sid = lax.axis_index("s")
        t = cid * 16 + sid
        iota = lax.iota(jnp.int32, 16)
        n0 = t * NPB

        def iz(i, carry):
            acc[pl.ds(i * 16, 16)] = jnp.full((16,), -jnp.inf, jnp.float32)
            return carry

        lax.fori_loop(0, (NPB + 1) * FEAT // 16, iz, 0)

        for h in range(2):
            pltpu.sync_copy(bb_hbm.at[pl.ds((h * 33 + t) * 16, 32)],
                            bbv.at[pl.ds(0, 32)])
            sv = bbv[pl.ds(0, 16)]
            evv = bbv[pl.ds(16, 16)]
            start = sv[0] + h * HALF
            end = evv[0] + h * HALF
            nch = lax.shift_right_logical(end - start + SSTEP - 1, 8)

            def chunk(c, carry, h=h, start=start, end=end):
                s = start + c * SSTEP
                cs = jnp.minimum(s, (h + 1) * HALF - SCH)
                cs = lax.shift_right_logical(cs, 4) * 16
                hi = jnp.minimum(s + SSTEP, end)
                pltpu.sync_copy(m_hbm.at[pl.ds(cs * FEAT, SCH * FEAT)], mbuf)
                pltpu.sync_copy(dst_hbm.at[pl.ds(cs, SCH)], dbuf)

                def vloop(v, carry2):
                    dv = dbuf[pl.ds(v * 16, 16)]
                    ev = cs + v * 16 + iota
                    okv = (ev >= s) & (ev < hi)
                    rowv = jnp.where(okv, dv - n0, NPB)
                    for kk in range(16):
                        lo = rowv[kk] * FEAT
                        mo = (v * 16 + kk) * FEAT
                        for j in range(8):
                            a = acc[pl.ds(lo + j * 16, 16)]
                            mv = mbuf[pl.ds(mo + j * 16, 16)]
                            acc[pl.ds(lo + j * 16, 16)] = jnp.maximum(a, mv)
                    return carry2

                lax.fori_loop(0, SCH // 16, vloop, 0)
                return carry

            lax.fori_loop(0, nch, chunk, 0)

        pltpu.sync_copy(acc.at[pl.ds(0, NPB * FEAT)],
                        out_hbm.at[pl.ds(n0 * FEAT, NPB * FEAT)])

    out = k(m_flat, dsts_perm, bb)
    return out.reshape(32 * NPB, FEAT)[:N]


BLKE = 2000


def _edge_body(ga_ref, gc_ref, w2_ref, out_ref):
    pre = jnp.maximum(ga_ref[...] + gc_ref[...], 0.0)
    out_ref[...] = jnp.dot(pre, w2_ref[...], preferred_element_type=jnp.float32)


def _edgemm(GA, GC, W2):
    grid = E // BLKE
    espec = pl.BlockSpec((BLKE, FEAT), lambda i: (i, 0))
    return pl.pallas_call(
        _edge_body,
        grid=(grid,),
        in_specs=[espec, espec, pl.BlockSpec((FEAT, FEAT), lambda i: (0, 0))],
        out_specs=espec,
        out_shape=jax.ShapeDtypeStruct((E, FEAT), jnp.float32),
    )(GA, GC, W2)


def _final_body(mc_ref, std_ref, z_ref, b2c_ref, out_ref):
    @pl.when(pl.program_id(0) == 0)
    def _():
        out_ref[0, 0] = 0.0

    m0 = mc_ref[:, 0:1]
    s = jnp.where(jnp.isfinite(m0), m0 + b2c_ref[0, 0], 0.0)
    score = s / (std_ref[...] + 1e-7)
    v = (score * std_ref[...] + z_ref[...]) ** 2
    out_ref[0, 0] += jnp.sum(v)

    @pl.when(pl.program_id(0) == pl.num_programs(0) - 1)
    def _():
        out_ref[0, 0] = out_ref[0, 0] * (1.0 / B)


def _final(Mc, std, z, b2c2):
    grid = N // NB
    return pl.pallas_call(
        _final_body,
        grid=(grid,),
        in_specs=[
            pl.BlockSpec((NB, FEAT), lambda i: (i, 0)),
            pl.BlockSpec((NB, 1), lambda i: (i, 0)),
            pl.BlockSpec((NB, 1), lambda i: (i, 0)),
            pl.BlockSpec((1, 1), lambda i: (0, 0), memory_space=pltpu.SMEM),
        ],
        out_specs=pl.BlockSpec((1, 1), lambda i: (0, 0),
                               memory_space=pltpu.SMEM),
        out_shape=jax.ShapeDtypeStruct((1, 1), jnp.float32),
    )(Mc, std, z, b2c2)


def kernel(x, edge_index, batch, pcs, random_t, z, We, be, W1a, b1a, W2a, b2a,
           W1b, b1b, W2b, b2b, W1c, b1c, W2c, b2c, fW, tW, tb, P1, pb1, P2,
           pb2, P3, pb3):
    src = edge_index[0]
    dst = edge_index[1]
    src_s, dst_s, bb = _bucket_partition(dst, src)
    batch2 = batch.reshape(N, 1)
    pcs_flat = pcs.reshape(N * NPTS, 3)
    r = lambda v: v.reshape(1, -1)

    emb = _pointnet(pcs_flat, P1, r(pb1), P2, r(pb2), P3, r(pb3))
    A, C, te, std = _node0(x, z, batch2, random_t, We, r(be), r(fW), tW,
                           r(tb), W1a, r(b1a), emb)

    for W1n, b1n, W2, b2 in ((W1b, b1b, W2a, b2a), (W1c, b1c, W2b, b2b)):
        GA = _gather_rows(A, dst_s)
        GC = _gather_rows(C, src_s)
        m = _edgemm(GA, GC, W2)
        M = _sc_segmax(m.reshape(E * FEAT), dst_s, bb)
        A, C = _nodel(M, r(b2), te, emb, W1n, r(b1n))

    GA = _gather_rows(A, dst_s)
    GC = _gather_rows(C, src_s)
    W2c_pad = jnp.pad(W2c, ((0, 0), (0, FEAT - W2c.shape[1])))
    mc = _edgemm(GA, GC, W2c_pad)
    Mc = _sc_segmax(mc.reshape(E * FEAT), dst_s, bb)
    loss = _final(Mc, std, z, b2c.reshape(1, 1))
    return loss.reshape(())
